# baseline (speedup 1.0000x reference)
import jax
import jax.numpy as jnp
from jax import lax
from jax.experimental import pallas as pl
from jax.experimental.pallas import tpu as pltpu
from jax.experimental.pallas import tpu_sc as plsc

B, T, H, V = 1024, 200, 128, 256
NC, NS = 2, 16
NW = NC * NS
BPW = B // NW


def _sc_body(idx_hbm, table_hbm, mel_hbm, out_hbm, idx_v, rows_v, bulk_sem, gat_sem):
    wid = lax.axis_index("s") * NC + lax.axis_index("c")
    base = wid * BPW

    bulk = pltpu.async_copy(
        mel_hbm.at[pl.ds(base, BPW)],
        out_hbm.at[pl.ds(base, BPW), pl.ds(1, T), :],
        bulk_sem,
    )

    pltpu.sync_copy(idx_hbm.at[pl.ds(base, BPW)], idx_v)
    pltpu.async_copy(table_hbm.at[idx_v], rows_v.at[:, 0, :], gat_sem).wait()

    pltpu.sync_copy(rows_v, out_hbm.at[pl.ds(base, BPW), pl.ds(0, 1), :])

    bulk.wait()


@jax.jit
def _run(idx, table, mel):
    mesh = plsc.VectorSubcoreMesh(
        core_axis_name="c", subcore_axis_name="s", num_cores=NC, num_subcores=NS
    )
    return pl.kernel(
        _sc_body,
        out_type=jax.ShapeDtypeStruct((B, T + 1, H), jnp.float32),
        mesh=mesh,
        scratch_types=[
            pltpu.VMEM((BPW,), jnp.int32),
            pltpu.VMEM((BPW, 1, H), jnp.float32),
            pltpu.SemaphoreType.DMA,
            pltpu.SemaphoreType.DMA,
        ],
        compiler_params=pltpu.CompilerParams(use_tc_tiling_on_sc=False),
    )(idx, table, mel)


def kernel(arranger_id, mel_db, table):
    idx = arranger_id.reshape(B).astype(jnp.int32)
    return _run(idx, table, mel_db)


# speedup vs baseline: 13.3864x; 13.3864x over previous
import jax
import jax.numpy as jnp
from jax import lax
from jax.experimental import pallas as pl
from jax.experimental.pallas import tpu as pltpu
from jax.experimental.pallas import tpu_sc as plsc

B, T, H, V = 1024, 200, 128, 256
NC, NS = 2, 16
NW = NC * NS
BPW = B // NW


CH = 2
NCH = BPW // CH


def _sc_body(idx_hbm, table_hbm, mel_hbm, out_hbm,
             idx_v, rows_v, buf0, buf1, rs0, rs1, ws0, ws1, gat_sem):
    wid = lax.axis_index("s") * NC + lax.axis_index("c")
    base = wid * BPW
    bufs, rsems, wsems = [buf0, buf1], [rs0, rs1], [ws0, ws1]

    pltpu.sync_copy(idx_hbm.at[pl.ds(base, BPW)], idx_v)
    pltpu.async_copy(table_hbm.at[idx_v], rows_v.at[:, 0, :], gat_sem).wait()

    emb_wr = pltpu.async_copy(
        rows_v, out_hbm.at[pl.ds(base, BPW), pl.ds(0, 1), :], gat_sem
    )

    rd = [None] * NCH
    wr = [None] * NCH
    rd[0] = pltpu.async_copy(mel_hbm.at[pl.ds(base, CH)], buf0, rs0)
    rd[1] = pltpu.async_copy(mel_hbm.at[pl.ds(base + CH, CH)], buf1, rs1)
    for g in range(NCH):
        b = g % 2
        rd[g].wait()
        wr[g] = pltpu.async_copy(
            bufs[b],
            out_hbm.at[pl.ds(base + g * CH, CH), pl.ds(1, T), :],
            wsems[b],
        )
        if g + 2 < NCH:
            wr[g].wait()
            rd[g + 2] = pltpu.async_copy(
                mel_hbm.at[pl.ds(base + (g + 2) * CH, CH)], bufs[b], rsems[b]
            )
    wr[NCH - 2].wait()
    wr[NCH - 1].wait()
    emb_wr.wait()


@jax.jit
def _run(idx, table, mel):
    mesh = plsc.VectorSubcoreMesh(
        core_axis_name="c", subcore_axis_name="s", num_cores=NC, num_subcores=NS
    )
    return pl.kernel(
        _sc_body,
        out_type=jax.ShapeDtypeStruct((B, T + 1, H), jnp.float32),
        mesh=mesh,
        scratch_types=[
            pltpu.VMEM((BPW,), jnp.int32),
            pltpu.VMEM((BPW, 1, H), jnp.float32),
            pltpu.VMEM((CH, T, H), jnp.float32),
            pltpu.VMEM((CH, T, H), jnp.float32),
            pltpu.SemaphoreType.DMA,
            pltpu.SemaphoreType.DMA,
            pltpu.SemaphoreType.DMA,
            pltpu.SemaphoreType.DMA,
            pltpu.SemaphoreType.DMA,
        ],
        compiler_params=pltpu.CompilerParams(use_tc_tiling_on_sc=False),
    )(idx, table, mel)


def kernel(arranger_id, mel_db, table):
    idx = arranger_id.reshape(B).astype(jnp.int32)
    return _run(idx, table, mel_db)


# speedup vs baseline: 13.4144x; 1.0021x over previous
import jax
import jax.numpy as jnp
from jax import lax
from jax.experimental import pallas as pl
from jax.experimental.pallas import tpu as pltpu
from jax.experimental.pallas import tpu_sc as plsc

B, T, H, V = 1024, 200, 128, 256
NC, NS = 2, 16
NW = NC * NS
BPW = B // NW


NBUF = 4


def _sc_body(idx_hbm, table_hbm, mel_hbm, out_hbm,
             idx_v, rows_v, bufs, rsems, wsems, gat_sem):
    wid = lax.axis_index("s") * NC + lax.axis_index("c")
    base = wid * BPW

    rd = [None] * BPW
    wr = [None] * BPW
    for g in range(NBUF):
        rd[g] = pltpu.async_copy(
            mel_hbm.at[pl.ds(base + g, 1)], bufs[g], rsems[g]
        )

    pltpu.sync_copy(idx_hbm.at[pl.ds(base, BPW)], idx_v)
    pltpu.async_copy(table_hbm.at[idx_v], rows_v.at[:, 0, :], gat_sem).wait()
    emb_wr = pltpu.async_copy(
        rows_v, out_hbm.at[pl.ds(base, BPW), pl.ds(0, 1), :], gat_sem
    )

    for g in range(BPW):
        b = g % NBUF
        rd[g].wait()
        wr[g] = pltpu.async_copy(
            bufs[b],
            out_hbm.at[pl.ds(base + g, 1), pl.ds(1, T), :],
            wsems[b],
        )
        if g + NBUF < BPW:
            wr[g].wait()
            rd[g + NBUF] = pltpu.async_copy(
                mel_hbm.at[pl.ds(base + g + NBUF, 1)], bufs[b], rsems[b]
            )
    for g in range(BPW - NBUF, BPW):
        wr[g].wait()
    emb_wr.wait()


@jax.jit
def _run(idx, table, mel):
    mesh = plsc.VectorSubcoreMesh(
        core_axis_name="c", subcore_axis_name="s", num_cores=NC, num_subcores=NS
    )
    return pl.kernel(
        _sc_body,
        out_type=jax.ShapeDtypeStruct((B, T + 1, H), jnp.float32),
        mesh=mesh,
        scratch_types=[
            pltpu.VMEM((BPW,), jnp.int32),
            pltpu.VMEM((BPW, 1, H), jnp.float32),
            [pltpu.VMEM((1, T, H), jnp.float32)] * NBUF,
            [pltpu.SemaphoreType.DMA] * NBUF,
            [pltpu.SemaphoreType.DMA] * NBUF,
            pltpu.SemaphoreType.DMA,
        ],
        compiler_params=pltpu.CompilerParams(use_tc_tiling_on_sc=False),
    )(idx, table, mel)


def kernel(arranger_id, mel_db, table):
    idx = arranger_id.reshape(B).astype(jnp.int32)
    return _run(idx, table, mel_db)


# speedup vs baseline: 20.5423x; 1.5314x over previous
import jax
import jax.numpy as jnp
from jax import lax
from jax.experimental import pallas as pl
from jax.experimental.pallas import tpu as pltpu
from jax.experimental.pallas import tpu_sc as plsc

B, T, H, V = 1024, 200, 128, 256
NC, NS = 2, 16
NW = NC * NS
BPW = B // NW
NBUF = 2


def _sc_body(idx_hbm, table_hbm, mel_hbm, out_hbm,
             idx_v, rows_v, bufs, rsems, wsems, gat_sem):
    wid = lax.axis_index("s") * NC + lax.axis_index("c")
    base = wid * BPW

    pltpu.sync_copy(idx_hbm.at[pl.ds(base, BPW)], idx_v)
    pltpu.async_copy(table_hbm.at[idx_v], rows_v, gat_sem).wait()

    rd = [None] * BPW
    wr = [None] * BPW
    for g in range(NBUF):
        rd[g] = pltpu.async_copy(
            mel_hbm.at[pl.ds(base + g, 1)], bufs[g].at[:, pl.ds(1, T), :], rsems[g]
        )
    for g in range(BPW):
        b = g % NBUF
        rd[g].wait()
        for j in range(H // 16):
            bufs[b][0, 0, pl.ds(j * 16, 16)] = rows_v[g, pl.ds(j * 16, 16)]
        wr[g] = pltpu.async_copy(
            bufs[b], out_hbm.at[pl.ds(base + g, 1)], wsems[b]
        )
        if g + NBUF < BPW:
            wr[g].wait()
            rd[g + NBUF] = pltpu.async_copy(
                mel_hbm.at[pl.ds(base + g + NBUF, 1)],
                bufs[b].at[:, pl.ds(1, T), :],
                rsems[b],
            )
    for g in range(BPW - NBUF, BPW):
        wr[g].wait()


@jax.jit
def _run(idx, table, mel):
    mesh = plsc.VectorSubcoreMesh(
        core_axis_name="c", subcore_axis_name="s", num_cores=NC, num_subcores=NS
    )
    return pl.kernel(
        _sc_body,
        out_type=jax.ShapeDtypeStruct((B, T + 1, H), jnp.float32),
        mesh=mesh,
        scratch_types=[
            pltpu.VMEM((BPW,), jnp.int32),
            pltpu.VMEM((BPW, H), jnp.float32),
            [pltpu.VMEM((1, T + 1, H), jnp.float32)] * NBUF,
            [pltpu.SemaphoreType.DMA] * NBUF,
            [pltpu.SemaphoreType.DMA] * NBUF,
            pltpu.SemaphoreType.DMA,
        ],
        compiler_params=pltpu.CompilerParams(use_tc_tiling_on_sc=True),
    )(idx, table, mel)


def kernel(arranger_id, mel_db, table):
    idx = arranger_id.reshape(B).astype(jnp.int32)
    return _run(idx, table, mel_db)
